# baseline (speedup 1.0000x reference)
import functools

import jax
import jax.numpy as jnp
from jax import lax
from jax.experimental import pallas as pl
from jax.experimental.pallas import tpu as pltpu
from jax.experimental.pallas import tpu_sc as plsc


_RB = 512
_CB = 2048
_SRB = 64


def _argmin_body(z_ref, c_ref, idx_ref, c16_ref, y2_ref):
    r = pl.program_id(0)
    v = c_ref.shape[0]

    @pl.when(r == 0)
    def _():
        cb = c_ref[...]
        c16_ref[...] = cb.astype(jnp.bfloat16)
        y2_ref[...] = jnp.sum(cb * cb, axis=1).reshape(1, v)

    zb32 = z_ref[...]
    x2 = jnp.sum(zb32 * zb32, axis=1, keepdims=True)
    zb = (-2.0 * zb32).astype(jnp.bfloat16)

    nlanes = 128

    def chunk_dot(k):
        cbk = c16_ref[pl.ds(k * _CB, _CB), :]
        return lax.dot_general(
            zb, cbk,
            dimension_numbers=(((1,), (1,)), ((), ())),
            preferred_element_type=jnp.float32,
        )

    def chunk_argmin(k, s):
        p = jcol = None
        for j in range(_CB // nlanes):
            y2j = y2_ref[:, pl.ds(k * _CB + j * nlanes, nlanes)]
            dj = (x2 + y2j) + s[:, j * nlanes:(j + 1) * nlanes]
            if j == 0:
                p = dj
                jcol = jnp.zeros(dj.shape, jnp.int32)
            else:
                lt = dj < p
                p = jnp.where(lt, dj, p)
                jcol = jnp.where(lt, j, jcol)
        m = jnp.min(p, axis=1, keepdims=True)
        lane = lax.broadcasted_iota(jnp.int32, p.shape, 1)
        key = jcol * nlanes + lane
        li = jnp.min(jnp.where(p == m, key, jnp.int32(2**30)),
                     axis=1, keepdims=True) + k * _CB
        return m, li

    nk = v // _CB
    bv = bi = None
    s_cur = chunk_dot(0)
    for k in range(nk):
        s_next = chunk_dot(k + 1) if k + 1 < nk else None
        m, li = chunk_argmin(k, s_cur)
        s_cur = s_next
        if k == 0:
            bv, bi = m, li
        else:
            better = m < bv
            bi = jnp.where(better, li, bi)
            bv = jnp.where(better, m, bv)
    idx_ref[...] = bi


def _argmin_call(z, codebook, sub):
    n, _ = z.shape
    v, k = codebook.shape
    return pl.pallas_call(
        _argmin_body,
        grid=(n // _RB,),
        in_specs=[
            pl.BlockSpec((_RB, k), lambda r: (r, sub)),
            pl.BlockSpec((v, k), lambda r: (0, 0)),
        ],
        out_specs=pl.BlockSpec((_RB, 1), lambda r: (r, 0)),
        out_shape=jax.ShapeDtypeStruct((n, 1), jnp.int32),
        scratch_shapes=[
            pltpu.VMEM((v, k), jnp.bfloat16),
            pltpu.VMEM((1, v), jnp.float32),
        ],
        compiler_params=pltpu.CompilerParams(
            dimension_semantics=("arbitrary",),
        ),
    )(z, codebook)


_MESH = None


def _sc_mesh():
    global _MESH
    if _MESH is None:
        _MESH = plsc.VectorSubcoreMesh(core_axis_name="c", subcore_axis_name="s")
    return _MESH


def _gather_first(table, idx):
    n = idx.shape[0]
    d = table.shape[1]
    info = plsc.get_sparse_core_info()
    nw = info.num_cores * info.num_subcores
    nc = info.num_cores
    per = n // nw

    @functools.partial(
        pl.kernel,
        mesh=_sc_mesh(),
        out_type=jax.ShapeDtypeStruct((n, d), jnp.float32),
        scratch_types=[
            pltpu.VMEM((per,), jnp.int32),
            pltpu.VMEM((per, d), jnp.float32),
            pltpu.SemaphoreType.DMA,
        ],
    )
    def gather_kernel(table_hbm, idx_hbm, out_hbm, idx_v, rows_v, sem):
        wid = lax.axis_index("s") * nc + lax.axis_index("c")
        base = wid * per
        pltpu.sync_copy(idx_hbm.at[pl.ds(base, per)], idx_v)
        pltpu.async_copy(table_hbm.at[idx_v], rows_v, sem).wait()
        pltpu.sync_copy(rows_v, out_hbm.at[pl.ds(base, per)])

    return gather_kernel(table, idx)


def _gather_merge(table, idx, q0):
    n = idx.shape[0]
    d = table.shape[1]
    info = plsc.get_sparse_core_info()
    nw = info.num_cores * info.num_subcores
    nc = info.num_cores
    per = n // (nw // 2)

    @functools.partial(
        pl.kernel,
        mesh=_sc_mesh(),
        out_type=jax.ShapeDtypeStruct((n, 2, d), jnp.float32),
        scratch_types=[
            pltpu.VMEM((per,), jnp.int32),
            pltpu.VMEM((per, d), jnp.float32),
            pltpu.SemaphoreType.DMA,
        ],
    )
    def merge_kernel(table_hbm, idx_hbm, q0_hbm, out_hbm, idx_v, rows_v, sem):
        wid = lax.axis_index("s") * nc + lax.axis_index("c")

        @pl.when(wid < nw // 2)
        def _():
            base = wid * per
            pltpu.sync_copy(idx_hbm.at[pl.ds(base, per)], idx_v)
            pltpu.async_copy(table_hbm.at[idx_v], rows_v, sem).wait()
            pltpu.sync_copy(rows_v, out_hbm.at[pl.ds(base, per), 1])

        @pl.when(wid >= nw // 2)
        def _():
            base = (wid - nw // 2) * per
            pltpu.sync_copy(q0_hbm.at[pl.ds(base, per)], rows_v)
            pltpu.sync_copy(rows_v, out_hbm.at[pl.ds(base, per), 0])

    return merge_kernel(table, idx, q0)


def kernel(embeddings, codebook0, codebook1):
    batch, seq, emb = embeddings.shape
    z = embeddings.reshape(-1, emb)

    idx0 = _argmin_call(z, codebook0, 0)
    q0 = _gather_first(codebook0, idx0.reshape(-1))
    idx1 = _argmin_call(z, codebook1, 1)
    out = _gather_merge(codebook1, idx1.reshape(-1), q0)

    all_idx = jnp.concatenate([idx0, idx1], axis=1)
    return (out.reshape(batch, seq, emb),
            all_idx.reshape(batch, seq, 2))


# speedup vs baseline: 1.0265x; 1.0265x over previous
import functools

import jax
import jax.numpy as jnp
from jax import lax
from jax.experimental import pallas as pl
from jax.experimental.pallas import tpu as pltpu
from jax.experimental.pallas import tpu_sc as plsc


_RB = 512
_CB = 2048
_SRB = 64


def _argmin_body(z_ref, c_ref, idx_ref, c16_ref, y2_ref):
    r = pl.program_id(0)
    v = c_ref.shape[0]

    @pl.when(r == 0)
    def _():
        cb = c_ref[...]
        c16_ref[...] = cb.astype(jnp.bfloat16)
        y2_ref[...] = jnp.sum(cb * cb, axis=1).reshape(1, v)

    zb32 = z_ref[...]
    x2 = jnp.sum(zb32 * zb32, axis=1, keepdims=True)
    zb = (-2.0 * zb32).astype(jnp.bfloat16)

    nlanes = 128

    def chunk_dot(k):
        cbk = c16_ref[pl.ds(k * _CB, _CB), :]
        return lax.dot_general(
            zb, cbk,
            dimension_numbers=(((1,), (1,)), ((), ())),
            preferred_element_type=jnp.float32,
        )

    def chunk_argmin(k, s):
        p = jcol = None
        for j in range(_CB // nlanes):
            y2j = y2_ref[:, pl.ds(k * _CB + j * nlanes, nlanes)]
            dj = (x2 + y2j) + s[:, j * nlanes:(j + 1) * nlanes]
            if j == 0:
                p = dj
                jcol = jnp.zeros(dj.shape, jnp.int32)
            else:
                lt = dj < p
                p = jnp.where(lt, dj, p)
                jcol = jnp.where(lt, j, jcol)
        m = jnp.min(p, axis=1, keepdims=True)
        lane = lax.broadcasted_iota(jnp.int32, p.shape, 1)
        key = jcol * nlanes + lane
        li = jnp.min(jnp.where(p == m, key, jnp.int32(2**30)),
                     axis=1, keepdims=True) + k * _CB
        return m, li

    nk = v // _CB
    bv = bi = None
    s_cur = chunk_dot(0)
    for k in range(nk):
        s_next = chunk_dot(k + 1) if k + 1 < nk else None
        m, li = chunk_argmin(k, s_cur)
        s_cur = s_next
        if k == 0:
            bv, bi = m, li
        else:
            better = m < bv
            bi = jnp.where(better, li, bi)
            bv = jnp.where(better, m, bv)
    idx_ref[...] = bi


def _argmin_call(z, codebook, sub):
    n, _ = z.shape
    v, k = codebook.shape
    return pl.pallas_call(
        _argmin_body,
        grid=(n // _RB,),
        in_specs=[
            pl.BlockSpec((_RB, k), lambda r: (r, sub)),
            pl.BlockSpec((v, k), lambda r: (0, 0)),
        ],
        out_specs=pl.BlockSpec((_RB, 1), lambda r: (r, 0)),
        out_shape=jax.ShapeDtypeStruct((n, 1), jnp.int32),
        scratch_shapes=[
            pltpu.VMEM((v, k), jnp.bfloat16),
            pltpu.VMEM((1, v), jnp.float32),
        ],
        compiler_params=pltpu.CompilerParams(
            dimension_semantics=("arbitrary",),
        ),
    )(z, codebook)


_MESH = None


def _sc_mesh():
    global _MESH
    if _MESH is None:
        _MESH = plsc.VectorSubcoreMesh(core_axis_name="c", subcore_axis_name="s")
    return _MESH


def _gather_both(table0, table1, idx0, idx1):
    n = idx0.shape[0]
    d = table0.shape[1]
    info = plsc.get_sparse_core_info()
    nw = info.num_cores * info.num_subcores
    nc = info.num_cores
    per = n // (nw // 2)

    @functools.partial(
        pl.kernel,
        mesh=_sc_mesh(),
        out_type=jax.ShapeDtypeStruct((n, 2, d), jnp.float32),
        scratch_types=[
            pltpu.VMEM((per,), jnp.int32),
            pltpu.VMEM((per, d), jnp.float32),
            pltpu.SemaphoreType.DMA,
        ],
    )
    def gather_kernel(t0_hbm, t1_hbm, i0_hbm, i1_hbm, out_hbm,
                      idx_v, rows_v, sem):
        wid = lax.axis_index("s") * nc + lax.axis_index("c")

        @pl.when(wid < nw // 2)
        def _():
            base = wid * per
            pltpu.sync_copy(i0_hbm.at[pl.ds(base, per)], idx_v)
            pltpu.async_copy(t0_hbm.at[idx_v], rows_v, sem).wait()
            pltpu.sync_copy(rows_v, out_hbm.at[pl.ds(base, per), 0])

        @pl.when(wid >= nw // 2)
        def _():
            base = (wid - nw // 2) * per
            pltpu.sync_copy(i1_hbm.at[pl.ds(base, per)], idx_v)
            pltpu.async_copy(t1_hbm.at[idx_v], rows_v, sem).wait()
            pltpu.sync_copy(rows_v, out_hbm.at[pl.ds(base, per), 1])

    return gather_kernel(table0, table1, idx0, idx1)


def kernel(embeddings, codebook0, codebook1):
    batch, seq, emb = embeddings.shape
    z = embeddings.reshape(-1, emb)

    idx0 = _argmin_call(z, codebook0, 0)
    idx1 = _argmin_call(z, codebook1, 1)
    out = _gather_both(codebook0, codebook1,
                       idx0.reshape(-1), idx1.reshape(-1))

    all_idx = jnp.concatenate([idx0, idx1], axis=1)
    return (out.reshape(batch, seq, emb),
            all_idx.reshape(batch, seq, 2))


# speedup vs baseline: 1.1090x; 1.0804x over previous
import functools

import jax
import jax.numpy as jnp
from jax import lax
from jax.experimental import pallas as pl
from jax.experimental.pallas import tpu as pltpu
from jax.experimental.pallas import tpu_sc as plsc


_RB = 576
_CB = 4096
_SRB = 64


def _argmin_body(z_ref, c_ref, idx_ref, c16_ref, y2_ref):
    r = pl.program_id(0)
    v = c_ref.shape[0]

    @pl.when(r == 0)
    def _():
        cb = c_ref[...]
        c16_ref[...] = cb.astype(jnp.bfloat16)
        cb3 = cb.reshape(v // 128, 128, cb.shape[1])
        y2_ref[...] = jnp.sum(cb3 * cb3, axis=2)

    zb32 = z_ref[...]
    x2 = jnp.sum(zb32 * zb32, axis=1, keepdims=True)
    zb = (-2.0 * zb32).astype(jnp.bfloat16)

    nlanes = 128

    def chunk_dot(k):
        cbk = c16_ref[pl.ds(k * _CB, _CB), :]
        return lax.dot_general(
            zb, cbk,
            dimension_numbers=(((1,), (1,)), ((), ())),
            preferred_element_type=jnp.float32,
        )

    def chunk_argmin(k, s):
        p = jcol = None
        for j in range(_CB // nlanes):
            g = k * (_CB // nlanes) + j
            y2j = y2_ref[pl.ds(g, 1), :]
            dj = (x2 + y2j) + s[:, j * nlanes:(j + 1) * nlanes]
            if j == 0:
                p = dj
                jcol = jnp.zeros(dj.shape, jnp.int32)
            else:
                lt = dj < p
                p = jnp.where(lt, dj, p)
                jcol = jnp.where(lt, j, jcol)
        m = jnp.min(p, axis=1, keepdims=True)
        lane = lax.broadcasted_iota(jnp.int32, p.shape, 1)
        key = jcol * nlanes + lane
        li = jnp.min(jnp.where(p == m, key, jnp.int32(2**30)),
                     axis=1, keepdims=True) + k * _CB
        return m, li

    nk = v // _CB
    bv = bi = None
    s_cur = chunk_dot(0)
    for k in range(nk):
        s_next = chunk_dot(k + 1) if k + 1 < nk else None
        m, li = chunk_argmin(k, s_cur)
        s_cur = s_next
        if k == 0:
            bv, bi = m, li
        else:
            better = m < bv
            bi = jnp.where(better, li, bi)
            bv = jnp.where(better, m, bv)
    idx_ref[...] = bi


def _argmin_call(z, codebook, sub):
    n, _ = z.shape
    v, k = codebook.shape
    return pl.pallas_call(
        _argmin_body,
        grid=(n // _RB,),
        in_specs=[
            pl.BlockSpec((_RB, k), lambda r: (r, sub)),
            pl.BlockSpec((v, k), lambda r: (0, 0)),
        ],
        out_specs=pl.BlockSpec((_RB, 1), lambda r: (r, 0)),
        out_shape=jax.ShapeDtypeStruct((n, 1), jnp.int32),
        scratch_shapes=[
            pltpu.VMEM((v, k), jnp.bfloat16),
            pltpu.VMEM((v // 128, 128), jnp.float32),
        ],
        compiler_params=pltpu.CompilerParams(
            dimension_semantics=("arbitrary",),
        ),
    )(z, codebook)


_MESH = None


def _sc_mesh():
    global _MESH
    if _MESH is None:
        _MESH = plsc.VectorSubcoreMesh(core_axis_name="c", subcore_axis_name="s")
    return _MESH


def _gather_call(table, gidx):
    b = gidx.shape[0]
    d = table.shape[1]
    info = plsc.get_sparse_core_info()
    nw = info.num_cores * info.num_subcores
    nc = info.num_cores
    b_per_w = b // nw

    half = b_per_w // 2

    @functools.partial(
        pl.kernel,
        mesh=_sc_mesh(),
        out_type=jax.ShapeDtypeStruct((b, d), jnp.float32),
        scratch_types=[
            pltpu.VMEM((half,), jnp.int32),
            pltpu.VMEM((half,), jnp.int32),
            pltpu.VMEM((half, d), jnp.float32),
            pltpu.VMEM((half, d), jnp.float32),
            pltpu.SemaphoreType.DMA,
            pltpu.SemaphoreType.DMA,
            pltpu.SemaphoreType.DMA,
        ],
    )
    def gather_kernel(table_hbm, idx_hbm, out_hbm,
                      ia_v, ib_v, bufa, bufb, sga, sgb, sw):
        wid = lax.axis_index("s") * nc + lax.axis_index("c")
        base = wid * b_per_w
        pltpu.sync_copy(idx_hbm.at[pl.ds(base, half)], ia_v)
        pltpu.sync_copy(idx_hbm.at[pl.ds(base + half, half)], ib_v)
        ha = pltpu.async_copy(table_hbm.at[ia_v], bufa, sga)
        hb = pltpu.async_copy(table_hbm.at[ib_v], bufb, sgb)
        ha.wait()
        wa = pltpu.async_copy(bufa, out_hbm.at[pl.ds(base, half)], sw)
        hb.wait()
        pltpu.sync_copy(bufb, out_hbm.at[pl.ds(base + half, half)])
        wa.wait()

    return gather_kernel(table, gidx)


def kernel(embeddings, codebook0, codebook1):
    batch, seq, emb = embeddings.shape
    z = embeddings.reshape(-1, emb)

    v = codebook0.shape[0]
    idx0 = _argmin_call(z, codebook0, 0)
    idx1 = _argmin_call(z, codebook1, 1)
    all_idx = jnp.concatenate([idx0, idx1], axis=1)

    table = jnp.concatenate([codebook0, codebook1], axis=0)
    gidx = (all_idx + jnp.array([0, v], jnp.int32)[None, :]).reshape(-1)
    quantized = _gather_call(table, gidx)

    return (quantized.reshape(batch, seq, emb),
            all_idx.reshape(batch, seq, 2))


# speedup vs baseline: 1.1116x; 1.0024x over previous
import functools

import jax
import jax.numpy as jnp
from jax import lax
from jax.experimental import pallas as pl
from jax.experimental.pallas import tpu as pltpu
from jax.experimental.pallas import tpu_sc as plsc


_RB = 576
_CB = 4096


def _argmin_body(z_ref, c_ref, idx_ref, c16_ref, y2_ref):
    r = pl.program_id(0)
    v = c_ref.shape[0]

    @pl.when(r == 0)
    def _():
        cb = c_ref[...]
        c16_ref[...] = cb.astype(jnp.bfloat16)
        cb3 = cb.reshape(v // 128, 128, cb.shape[1])
        y2_ref[...] = jnp.sum(cb3 * cb3, axis=2)

    zb32 = z_ref[...]
    x2 = jnp.sum(zb32 * zb32, axis=1, keepdims=True)
    zb = (-2.0 * zb32).astype(jnp.bfloat16)

    nlanes = 128

    def chunk_dot(k):
        cbk = c16_ref[pl.ds(k * _CB, _CB), :]
        return lax.dot_general(
            zb, cbk,
            dimension_numbers=(((1,), (1,)), ((), ())),
            preferred_element_type=jnp.float32,
        )

    def chunk_argmin(k, s):
        p = jcol = None
        for j in range(_CB // nlanes):
            g = k * (_CB // nlanes) + j
            y2j = y2_ref[pl.ds(g, 1), :]
            dj = (x2 + y2j) + s[:, j * nlanes:(j + 1) * nlanes]
            if j == 0:
                p = dj
                jcol = jnp.zeros(dj.shape, jnp.int32)
            else:
                lt = dj < p
                p = jnp.where(lt, dj, p)
                jcol = jnp.where(lt, j, jcol)
        m = jnp.min(p, axis=1, keepdims=True)
        lane = lax.broadcasted_iota(jnp.int32, p.shape, 1)
        key = jcol * nlanes + lane
        li = jnp.min(jnp.where(p == m, key, jnp.int32(2**30)),
                     axis=1, keepdims=True) + k * _CB
        return m, li

    nk = v // _CB
    bv = bi = None
    s_cur = chunk_dot(0)
    for k in range(nk):
        s_next = chunk_dot(k + 1) if k + 1 < nk else None
        m, li = chunk_argmin(k, s_cur)
        s_cur = s_next
        if k == 0:
            bv, bi = m, li
        else:
            better = m < bv
            bi = jnp.where(better, li, bi)
            bv = jnp.where(better, m, bv)
    idx_ref[...] = bi


def _argmin_call(z, codebook, sub):
    n, _ = z.shape
    v, k = codebook.shape
    return pl.pallas_call(
        _argmin_body,
        grid=(n // _RB,),
        in_specs=[
            pl.BlockSpec((_RB, k), lambda r: (r, sub)),
            pl.BlockSpec((v, k), lambda r: (0, 0)),
        ],
        out_specs=pl.BlockSpec((_RB, 1), lambda r: (r, 0)),
        out_shape=jax.ShapeDtypeStruct((n, 1), jnp.int32),
        scratch_shapes=[
            pltpu.VMEM((v, k), jnp.bfloat16),
            pltpu.VMEM((v // 128, 128), jnp.float32),
        ],
        compiler_params=pltpu.CompilerParams(
            dimension_semantics=("arbitrary",),
        ),
    )(z, codebook)


_MESH = None


def _sc_mesh():
    global _MESH
    if _MESH is None:
        _MESH = plsc.VectorSubcoreMesh(core_axis_name="c", subcore_axis_name="s")
    return _MESH


def _gather_call(table, gidx):
    b = gidx.shape[0]
    d = table.shape[1]
    info = plsc.get_sparse_core_info()
    nw = info.num_cores * info.num_subcores
    nc = info.num_cores
    b_per_w = b // nw

    half = b_per_w // 2

    @functools.partial(
        pl.kernel,
        mesh=_sc_mesh(),
        out_type=jax.ShapeDtypeStruct((b, d), jnp.float32),
        scratch_types=[
            pltpu.VMEM((half,), jnp.int32),
            pltpu.VMEM((half,), jnp.int32),
            pltpu.VMEM((half, d), jnp.float32),
            pltpu.VMEM((half, d), jnp.float32),
            pltpu.SemaphoreType.DMA,
            pltpu.SemaphoreType.DMA,
            pltpu.SemaphoreType.DMA,
        ],
    )
    def gather_kernel(table_hbm, idx_hbm, out_hbm,
                      ia_v, ib_v, bufa, bufb, sga, sgb, sw):
        wid = lax.axis_index("s") * nc + lax.axis_index("c")
        base = wid * b_per_w
        pltpu.sync_copy(idx_hbm.at[pl.ds(base, half)], ia_v)
        pltpu.sync_copy(idx_hbm.at[pl.ds(base + half, half)], ib_v)
        ha = pltpu.async_copy(table_hbm.at[ia_v], bufa, sga)
        hb = pltpu.async_copy(table_hbm.at[ib_v], bufb, sgb)
        ha.wait()
        wa = pltpu.async_copy(bufa, out_hbm.at[pl.ds(base, half)], sw)
        hb.wait()
        pltpu.sync_copy(bufb, out_hbm.at[pl.ds(base + half, half)])
        wa.wait()

    return gather_kernel(table, gidx)


def kernel(embeddings, codebook0, codebook1):
    batch, seq, emb = embeddings.shape
    z = embeddings.reshape(-1, emb)

    v = codebook0.shape[0]
    idx0 = _argmin_call(z, codebook0, 0)
    idx1 = _argmin_call(z, codebook1, 1)
    all_idx = jnp.concatenate([idx0, idx1], axis=1)

    table = jnp.concatenate([codebook0, codebook1], axis=0)
    gidx = (all_idx + jnp.array([0, v], jnp.int32)[None, :]).reshape(-1)
    quantized = _gather_call(table, gidx)

    return (quantized.reshape(batch, seq, emb),
            all_idx.reshape(batch, seq, 2))
